# Initial kernel scaffold; baseline (speedup 1.0000x reference)
import jax
import jax.numpy as jnp
from jax.experimental import pallas as pl


def kernel(log_probs, trg):
    raise NotImplementedError("write your pallas kernel here")


# speedup vs baseline: 1.0983x; 1.0983x over previous
import functools
import math

import jax
import jax.numpy as jnp
from jax import lax
from jax.experimental import pallas as pl
from jax.experimental.pallas import tpu as pltpu
from jax.experimental.pallas import tpu_sc as plsc

PAD = 0
SM = 0.1

NC = 2
NS = 16
L = 16
NW = NC * NS


def _sc_gather_corr(lp16, tflat, n_rows, vocab, c_const, w_t):
    rows_per_w = n_rows // NW
    mesh = plsc.VectorSubcoreMesh(core_axis_name="c", subcore_axis_name="s")

    @functools.partial(
        pl.kernel,
        out_type=jax.ShapeDtypeStruct((n_rows,), jnp.float32),
        mesh=mesh,
        scratch_types=[
            pltpu.VMEM((L,), jnp.int32),
            pltpu.VMEM((L,), jnp.int32),
            pltpu.VMEM((L, 128), jnp.float32),
            pltpu.VMEM((L,), jnp.float32),
            pltpu.SemaphoreType.DMA,
        ],
    )
    def body(lp_hbm, trg_hbm, out_hbm, t_v, idx_v, rows_v, corr_v, sem):
        wid = lax.axis_index("s") * NC + lax.axis_index("c")
        base = wid * rows_per_w
        pltpu.sync_copy(trg_hbm.at[pl.ds(base, rows_per_w)], t_v)
        t = t_v[...]
        r = base + lax.iota(jnp.int32, L)
        flat = r * vocab + t
        idx_v[...] = lax.shift_right_logical(flat, 7)
        pltpu.async_copy(lp_hbm.at[idx_v], rows_v, sem).wait()
        lane = lax.iota(jnp.int32, L)
        off = lax.bitwise_and(flat, 127)
        kv = lax.shift_right_logical(off, 4)
        lv = lax.bitwise_and(off, 15)
        dnums = lax.GatherDimensionNumbers(
            offset_dims=(), collapsed_slice_dims=(0,), start_index_map=(0,))
        tval = jnp.zeros((L,), jnp.float32)
        for i in range(L):
            row_hit = lane == i
            for k in range(128 // L):
                g = lax.gather(rows_v[i, k * L:(k + 1) * L], lv[:, None],
                               dnums, slice_sizes=(1,),
                               mode=lax.GatherScatterMode.PROMISE_IN_BOUNDS)
                tval = jnp.where(row_hit & (kv == k), g, tval)
        corr = jnp.where(t != PAD,
                         jnp.float32(c_const) + jnp.float32(w_t) * tval,
                         jnp.float32(0.0))
        corr_v[...] = corr
        pltpu.sync_copy(corr_v, out_hbm.at[pl.ds(base, rows_per_w)])

    return body(lp16, tflat)


def _tc_body(nb, vb, vocab, eps, lp_ref, trg_ref, corr_ref, out_ref):
    j = pl.program_id(0)
    m = (trg_ref[...] != PAD).astype(jnp.float32)

    @pl.when(j == 0)
    def _():
        zsum = jnp.sum(lp_ref[:, 0:1] * m)
        csum = jnp.sum(corr_ref[...])
        out_ref[0, 0] = csum + jnp.float32(eps) * zsum

    @pl.when(j < nb - 1)
    def _():
        out_ref[0, 0] -= jnp.float32(eps) * jnp.sum(lp_ref[...] * m)

    @pl.when(j == nb - 1)
    def _():
        col = (nb - 1) * vb + lax.broadcasted_iota(jnp.int32, lp_ref.shape, 1)
        x = jnp.where(col < vocab, lp_ref[...], 0.0)
        out_ref[0, 0] -= jnp.float32(eps) * jnp.sum(x * m)


def kernel(log_probs, trg):
    vocab = log_probs.shape[-1]
    lp = log_probs.reshape(-1, vocab)
    n = lp.shape[0]
    tflat = trg.reshape(-1).astype(jnp.int32)

    eps = SM / (vocab - 2)
    c_const = SM * math.log(eps) + (1.0 - SM) * math.log(1.0 - SM)
    w_t = eps - (1.0 - SM)

    corr = _sc_gather_corr(lp.reshape(-1, 128), tflat, n, vocab, c_const, w_t)

    vb = 4096
    nb = (vocab + vb - 1) // vb
    total = pl.pallas_call(
        functools.partial(_tc_body, nb, vb, vocab, eps),
        grid=(nb,),
        in_specs=[
            pl.BlockSpec((n, vb), lambda j: (0, j)),
            pl.BlockSpec((n, 1), lambda j: (0, 0)),
            pl.BlockSpec((4, 128), lambda j: (0, 0)),
        ],
        out_specs=pl.BlockSpec(memory_space=pltpu.SMEM),
        out_shape=jax.ShapeDtypeStruct((1, 1), jnp.float32),
    )(lp, tflat.reshape(n, 1), corr.reshape(4, 128))
    return total[0, 0]


# speedup vs baseline: 4.2588x; 3.8778x over previous
import functools
import math

import jax
import jax.numpy as jnp
from jax import lax
from jax.experimental import pallas as pl
from jax.experimental.pallas import tpu as pltpu
from jax.experimental.pallas import tpu_sc as plsc

PAD = 0
SM = 0.1

NC = 2
NS = 16
L = 16
NW = NC * NS


def _sc_gather_corr(lp, tflat, n_rows, vocab, c_const, w_t):
    rows_per_w = n_rows // NW
    mesh = plsc.VectorSubcoreMesh(core_axis_name="c", subcore_axis_name="s")

    @functools.partial(
        pl.kernel,
        out_type=jax.ShapeDtypeStruct((n_rows,), jnp.float32),
        mesh=mesh,
        scratch_types=[
            pltpu.VMEM((L,), jnp.int32),
            pltpu.VMEM((L, 8, 128), jnp.float32),
            pltpu.VMEM((L,), jnp.float32),
            pltpu.SemaphoreType.DMA,
        ],
    )
    def body(lp_hbm, trg_hbm, out_hbm, t_v, win_v, corr_v, sem):
        wid = lax.axis_index("s") * NC + lax.axis_index("c")
        base = wid * rows_per_w
        pltpu.sync_copy(trg_hbm.at[pl.ds(base, rows_per_w)], t_v)
        t = t_v[...]
        descs = []
        for i in range(L):
            c0 = pl.multiple_of(
                lax.shift_left(lax.shift_right_logical(t[i], 7), 7), 128)
            r0 = base + (i // 8) * 8
            descs.append(pltpu.async_copy(
                lp_hbm.at[pl.ds(r0, 8), pl.ds(c0, 128)], win_v.at[i], sem))
        for d in descs:
            d.wait()
        lane = lax.iota(jnp.int32, L)
        dnums = lax.GatherDimensionNumbers(
            offset_dims=(), collapsed_slice_dims=(0,), start_index_map=(0,))
        tval = jnp.zeros((L,), jnp.float32)
        for i in range(L):
            off = lax.bitwise_and(t[i], 127)
            sel = jnp.zeros((L,), jnp.float32)
            for k in range(128 // L):
                hit = (lane + k * L) == off
                sel = jnp.where(hit, win_v[i, i % 8, k * L:(k + 1) * L], sel)
            g = lax.gather(sel, jnp.full((L, 1), lax.bitwise_and(off, L - 1),
                                         jnp.int32),
                           dnums, slice_sizes=(1,),
                           mode=lax.GatherScatterMode.PROMISE_IN_BOUNDS)
            tval = jnp.where(lane == i, g, tval)
        corr = jnp.where(t != PAD,
                         jnp.float32(c_const) + jnp.float32(w_t) * tval,
                         jnp.float32(0.0))
        corr_v[...] = corr
        pltpu.sync_copy(corr_v, out_hbm.at[pl.ds(base, rows_per_w)])

    return body(lp, tflat)


def _tc_body(nb, vb, vocab, eps, lp_ref, trg_ref, corr_ref, out_ref):
    j = pl.program_id(0)
    m = (trg_ref[...] != PAD).astype(jnp.float32)

    @pl.when(j == 0)
    def _():
        zsum = jnp.sum(lp_ref[:, 0:1] * m)
        csum = jnp.sum(corr_ref[...])
        out_ref[0, 0] = csum + jnp.float32(eps) * zsum

    @pl.when(j < nb - 1)
    def _():
        out_ref[0, 0] -= jnp.float32(eps) * jnp.sum(lp_ref[...] * m)

    @pl.when(j == nb - 1)
    def _():
        col = (nb - 1) * vb + lax.broadcasted_iota(jnp.int32, lp_ref.shape, 1)
        x = jnp.where(col < vocab, lp_ref[...], 0.0)
        out_ref[0, 0] -= jnp.float32(eps) * jnp.sum(x * m)


def kernel(log_probs, trg):
    vocab = log_probs.shape[-1]
    lp = log_probs.reshape(-1, vocab)
    n = lp.shape[0]
    tflat = trg.reshape(-1).astype(jnp.int32)

    eps = SM / (vocab - 2)
    c_const = SM * math.log(eps) + (1.0 - SM) * math.log(1.0 - SM)
    w_t = eps - (1.0 - SM)

    corr = _sc_gather_corr(lp, tflat, n, vocab, c_const, w_t)

    vb = 4096
    nb = (vocab + vb - 1) // vb
    total = pl.pallas_call(
        functools.partial(_tc_body, nb, vb, vocab, eps),
        grid=(nb,),
        in_specs=[
            pl.BlockSpec((n, vb), lambda j: (0, j)),
            pl.BlockSpec((n, 1), lambda j: (0, 0)),
            pl.BlockSpec((4, 128), lambda j: (0, 0)),
        ],
        out_specs=pl.BlockSpec(memory_space=pltpu.SMEM),
        out_shape=jax.ShapeDtypeStruct((1, 1), jnp.float32),
    )(lp, tflat.reshape(n, 1), corr.reshape(4, 128))
    return total[0, 0]


# speedup vs baseline: 5.0740x; 1.1914x over previous
import functools
import math

import jax
import jax.numpy as jnp
from jax import lax
from jax.experimental import pallas as pl
from jax.experimental.pallas import tpu as pltpu
from jax.experimental.pallas import tpu_sc as plsc

PAD = 0
SM = 0.1

NC = 2
NS = 16
L = 16
NW = NC * NS


def _sc_gather_corr(lp, tflat, n_rows, vocab, c_const, w_t):
    rows_per_w = n_rows // NW
    mesh = plsc.VectorSubcoreMesh(core_axis_name="c", subcore_axis_name="s")

    @functools.partial(
        pl.kernel,
        out_type=jax.ShapeDtypeStruct((n_rows,), jnp.float32),
        mesh=mesh,
        scratch_types=[
            pltpu.VMEM((L,), jnp.int32),
            pltpu.VMEM((L, 8, 128), jnp.float32),
            pltpu.VMEM((L,), jnp.float32),
            pltpu.SemaphoreType.DMA,
        ],
    )
    def body(lp_hbm, trg_hbm, out_hbm, t_v, win_v, corr_v, sem):
        wid = lax.axis_index("s") * NC + lax.axis_index("c")
        base = wid * rows_per_w
        pltpu.sync_copy(trg_hbm.at[pl.ds(base, rows_per_w)], t_v)
        t = t_v[...]
        descs = []
        for i in range(L):
            c0 = pl.multiple_of(
                lax.shift_left(lax.shift_right_logical(t[i], 7), 7), 128)
            r0 = base + (i // 8) * 8
            descs.append(pltpu.async_copy(
                lp_hbm.at[pl.ds(r0, 8), pl.ds(c0, 128)], win_v.at[i], sem))
        for d in descs:
            d.wait()
        lane = lax.iota(jnp.int32, L)
        dnums = lax.GatherDimensionNumbers(
            offset_dims=(), collapsed_slice_dims=(0,), start_index_map=(0,))
        tval = jnp.zeros((L,), jnp.float32)
        for i in range(L):
            off = lax.bitwise_and(t[i], 127)
            sel = jnp.zeros((L,), jnp.float32)
            for k in range(128 // L):
                hit = (lane + k * L) == off
                sel = jnp.where(hit, win_v[i, i % 8, k * L:(k + 1) * L], sel)
            g = lax.gather(sel, jnp.full((L, 1), lax.bitwise_and(off, L - 1),
                                         jnp.int32),
                           dnums, slice_sizes=(1,),
                           mode=lax.GatherScatterMode.PROMISE_IN_BOUNDS)
            tval = jnp.where(lane == i, g, tval)
        corr = jnp.where(t != PAD,
                         jnp.float32(c_const) + jnp.float32(w_t) * tval,
                         jnp.float32(0.0))
        corr_v[...] = corr
        pltpu.sync_copy(corr_v, out_hbm.at[pl.ds(base, rows_per_w)])

    return body(lp, tflat)


def _tc_body(nb, vb, vocab, eps, lp_ref, trg_ref, corr_ref, out_ref, acc_ref):
    j = pl.program_id(0)

    def accum(x):
        s = x[:, 0:128]
        for k in range(1, vb // 128):
            s = s + x[:, 128 * k:128 * (k + 1)]
        acc_ref[...] += s

    @pl.when(j == 0)
    def _():
        acc_ref[...] = jnp.zeros_like(acc_ref)
        m = (trg_ref[...] != PAD).astype(jnp.float32)
        zsum = jnp.sum(lp_ref[:, 0:1] * m)
        csum = jnp.sum(corr_ref[...])
        out_ref[0, 0] = csum + jnp.float32(eps) * zsum

    @pl.when(j < nb - 1)
    def _():
        accum(lp_ref[...])

    @pl.when(j == nb - 1)
    def _():
        col = (nb - 1) * vb + lax.broadcasted_iota(jnp.int32, lp_ref.shape, 1)
        accum(jnp.where(col < vocab, lp_ref[...], 0.0))
        m = (trg_ref[...] != PAD).astype(jnp.float32)
        out_ref[0, 0] -= jnp.float32(eps) * jnp.sum(acc_ref[...] * m)


def kernel(log_probs, trg):
    vocab = log_probs.shape[-1]
    lp = log_probs.reshape(-1, vocab)
    n = lp.shape[0]
    tflat = trg.reshape(-1).astype(jnp.int32)

    eps = SM / (vocab - 2)
    c_const = SM * math.log(eps) + (1.0 - SM) * math.log(1.0 - SM)
    w_t = eps - (1.0 - SM)

    corr = _sc_gather_corr(lp, tflat, n, vocab, c_const, w_t)

    vb = 4096
    nb = (vocab + vb - 1) // vb
    total = pl.pallas_call(
        functools.partial(_tc_body, nb, vb, vocab, eps),
        grid=(nb,),
        in_specs=[
            pl.BlockSpec((n, vb), lambda j: (0, j)),
            pl.BlockSpec((n, 1), lambda j: (0, 0)),
            pl.BlockSpec((4, 128), lambda j: (0, 0)),
        ],
        out_specs=pl.BlockSpec(memory_space=pltpu.SMEM),
        out_shape=jax.ShapeDtypeStruct((1, 1), jnp.float32),
        scratch_shapes=[pltpu.VMEM((n, 128), jnp.float32)],
    )(lp, tflat.reshape(n, 1), corr.reshape(4, 128))
    return total[0, 0]
